# baseline (speedup 1.0000x reference)
import functools

import jax
import jax.numpy as jnp
from jax import lax
from jax.experimental import pallas as pl
from jax.experimental.pallas import tpu as pltpu
from jax.experimental.pallas import tpu_sc as plsc

NC = 2
NS = 16
L = 16
NW = NC * NS


def _sc_mesh():
    return plsc.VectorSubcoreMesh(
        core_axis_name="c", subcore_axis_name="s", num_cores=NC, num_subcores=NS
    )


@functools.lru_cache(maxsize=None)
def _deg_fn(E: int, N: int):
    EPW = E // NW
    NP = _pad16x8(N)
    assert EPW * NW == E and EPW % L == 0 and N % L == 0

    @functools.partial(
        pl.kernel,
        out_type=jax.ShapeDtypeStruct((NW, NP), jnp.float32),
        mesh=_sc_mesh(),
        compiler_params=pltpu.CompilerParams(needs_layout_passes=False),
        scratch_types=[
            pltpu.VMEM((EPW,), jnp.int32),
            pltpu.VMEM((NP,), jnp.float32),
        ],
    )
    def k(ei_hbm, out_hbm, idx_v, hist_v):
        cid = lax.axis_index("c")
        sid = lax.axis_index("s")
        wid = cid * NS + sid
        pltpu.sync_copy(ei_hbm.at[pl.ds(E + wid * EPW, EPW)], idx_v)
        zeros = jnp.zeros((L,), jnp.float32)

        def zbody(i, carry):
            hist_v[pl.ds(i * L, L)] = zeros
            return carry

        lax.fori_loop(0, NP // L, zbody, 0)
        ones = jnp.full((L,), 1.0, jnp.float32)

        def body(i, carry):
            ids = idx_v[pl.ds(i * L, L)]
            plsc.addupdate_scatter(hist_v, [ids], ones)
            return carry

        lax.fori_loop(0, EPW // L, body, 0)
        pltpu.sync_copy(hist_v, out_hbm.at[wid])

    return k


def _pad16x8(n):
    q = NS * 40
    return ((n + q - 1) // q) * q


@functools.lru_cache(maxsize=None)
def _agg_fn(N: int, D: int, E: int, K: int, ZR: int, NB: int = 5, LA: int = 3,
            side: bool = False, spt: bool = False):
    EPW = E // NW
    NCH = EPW // K
    NP = _pad16x8(N)
    RPT = NP // NS
    assert NCH * K == EPW and RPT % ZR == 0 and ZR % 8 == 0 and D % L == 0
    assert NCH % NB == 0 and LA < NB and K % 8 == 0

    out_shape = (NP, NC * D) if side else (NC * NP, D)

    @functools.partial(
        pl.kernel,
        out_type=jax.ShapeDtypeStruct(out_shape, jnp.float32),
        mesh=_sc_mesh(),
        compiler_params=pltpu.CompilerParams(
            needs_layout_passes=False, use_tc_tiling_on_sc=False
        ),
        scratch_types=[
            pltpu.VMEM((EPW,), jnp.int32),
            pltpu.VMEM((NCH, K), jnp.int32),
            pltpu.VMEM((NB, K, D), jnp.float32),
            pltpu.VMEM((ZR, D), jnp.float32),
            pltpu.VMEM_SHARED((NP, D), jnp.float32),
            pltpu.VMEM_SHARED((NP, D) if spt else (8, 8), jnp.float32),
            pltpu.SemaphoreType.DMA,
            pltpu.SemaphoreType.DMA,
            [pltpu.SemaphoreType.DMA] * NB,
            [pltpu.SemaphoreType.DMA] * NB,
        ],
    )
    def k(table, ei_hbm, out_hbm, sidx_v, didx, rows, zbuf, acc, table_sh,
          isem, dsem, gsems, ssems):
        cid = lax.axis_index("c")
        sid = lax.axis_index("s")
        wid = cid * NS + sid
        ebase = wid * EPW

        pltpu.async_copy(ei_hbm.at[pl.ds(ebase, EPW)], sidx_v, isem)

        def dfetch(j, carry):
            pltpu.async_copy(
                ei_hbm.at[pl.ds(E + ebase + j * K, K)], didx.at[j], dsem
            )
            return carry

        lax.fori_loop(0, NCH, dfetch, 0)

        zeros = jnp.zeros((L,), jnp.float32)
        vpr = D // L

        def zb(t, carry):
            zbuf[t // vpr, pl.ds((t % vpr) * L, L)] = zeros
            return carry

        lax.fori_loop(0, ZR * vpr, zb, 0)
        for j in range(RPT // ZR):
            pltpu.sync_copy(zbuf, acc.at[pl.ds(sid * RPT + j * ZR, ZR)])
        if spt:
            pltpu.sync_copy(
                table.at[pl.ds(sid * RPT, RPT)],
                table_sh.at[pl.ds(sid * RPT, RPT)],
            )
        pltpu.make_async_copy(ei_hbm.at[pl.ds(ebase, EPW)], sidx_v, isem).wait()

        def dwait(j, carry):
            pltpu.make_async_copy(
                ei_hbm.at[pl.ds(E + ebase, K)], didx.at[0], dsem
            ).wait()
            return carry

        lax.fori_loop(0, NCH, dwait, 0)
        plsc.subcore_barrier()

        gsrc = table_sh if spt else table

        def start_gather(j, b):
            pltpu.async_copy(
                gsrc.at[sidx_v.at[pl.ds(j * K, K)]], rows.at[b], gsems[b]
            )

        def wait_gather(b):
            pltpu.make_async_copy(
                table.at[pl.ds(0, K)], rows.at[b], gsems[b]
            ).wait()

        def start_scatter(j, b):
            pltpu.async_copy(rows.at[b], acc.at[didx.at[j]], ssems[b], add=True)

        def wait_scatter(b):
            pltpu.make_async_copy(rows.at[b], acc.at[pl.ds(0, K)], ssems[b]).wait()

        for b in range(LA):
            start_gather(b, b)

        def body(i, carry):
            j0 = i * NB
            for b in range(NB):
                j = j0 + b
                wait_gather(b)
                start_scatter(j, b)
                jn = j + LA
                bn = (b + LA) % NB

                @pl.when(jn < NCH)
                def _():
                    @pl.when(jn >= NB)
                    def _():
                        wait_scatter(bn)

                    start_gather(jn, bn)

            return carry

        lax.fori_loop(0, NCH // NB, body, 0)
        for b in range(NB):
            wait_scatter(b)
        plsc.subcore_barrier()
        if side:
            pltpu.sync_copy(
                acc.at[pl.ds(sid * RPT, RPT)],
                out_hbm.at[pl.ds(sid * RPT, RPT), pl.ds(cid * D, D)],
            )
        else:
            pltpu.sync_copy(
                acc.at[pl.ds(sid * RPT, RPT)],
                out_hbm.at[pl.ds(cid * NP + sid * RPT, RPT)],
            )

    return k


def _dinv(hist):
    NWh, N = hist.shape

    def body(h_ref, o_ref):
        s = jnp.sum(h_ref[...], axis=0, keepdims=True) + 1.0
        o_ref[...] = lax.rsqrt(s)

    return pl.pallas_call(
        body, out_shape=jax.ShapeDtypeStruct((1, N), jnp.float32)
    )(hist)


def _matmul_prescale(x, W, dinv_col, rb=2000):
    N, Din = x.shape
    NP = _pad16x8(N)
    Dout = W.shape[1]

    def body(x_ref, w_ref, dv_ref, o_ref):
        h = jnp.dot(x_ref[...], w_ref[...], preferred_element_type=jnp.float32)
        o_ref[...] = h * dv_ref[...]

    return pl.pallas_call(
        body,
        grid=(N // rb,),
        in_specs=[
            pl.BlockSpec((rb, Din), lambda i: (i, 0)),
            pl.BlockSpec((Din, Dout), lambda i: (0, 0)),
            pl.BlockSpec((rb, 1), lambda i: (i, 0)),
        ],
        out_specs=pl.BlockSpec((rb, Dout), lambda i: (i, 0)),
        out_shape=jax.ShapeDtypeStruct((NP, Dout), jnp.float32),
    )(x, W, dinv_col)


def _combine_next(agg, scaled, dinv_col, b_row, Wn, rb=2048):
    NP, D = scaled.shape
    npb = NP // rb
    Dn = Wn.shape[1]

    def body(p0_ref, p1_ref, s_ref, dv_ref, b_ref, w_ref, o_ref):
        agg_ = (p0_ref[...] + p1_ref[...] + s_ref[...]) * dv_ref[...]
        act = jnp.tanh(agg_ + b_ref[...])
        h = jnp.dot(act, w_ref[...], preferred_element_type=jnp.float32)
        o_ref[...] = h * dv_ref[...]

    return pl.pallas_call(
        body,
        grid=(npb,),
        in_specs=[
            pl.BlockSpec((rb, D), lambda i: (i, 0)),
            pl.BlockSpec((rb, D), lambda i: (npb + i, 0)),
            pl.BlockSpec((rb, D), lambda i: (i, 0)),
            pl.BlockSpec((rb, 1), lambda i: (i, 0)),
            pl.BlockSpec((1, D), lambda i: (0, 0)),
            pl.BlockSpec((D, Dn), lambda i: (0, 0)),
        ],
        out_specs=pl.BlockSpec((rb, Dn), lambda i: (i, 0)),
        out_shape=jax.ShapeDtypeStruct((NP, Dn), jnp.float32),
    )(agg, agg, scaled, dinv_col, b_row, Wn)


def _combine_sbs(agg, scaled, dinv_col, b_row, Wn, rb=2048):
    NP, D = scaled.shape
    npb = NP // rb
    Dn = Wn.shape[1]

    def body(a_ref, s_ref, dv_ref, b_ref, w_ref, o_ref):
        a = a_ref[...]
        agg_ = (a[:, :D] + a[:, D:] + s_ref[...]) * dv_ref[...]
        act = jnp.tanh(agg_ + b_ref[...])
        h = jnp.dot(act, w_ref[...], preferred_element_type=jnp.float32)
        o_ref[...] = h * dv_ref[...]

    return pl.pallas_call(
        body,
        grid=(npb,),
        in_specs=[
            pl.BlockSpec((rb, 2 * D), lambda i: (i, 0)),
            pl.BlockSpec((rb, D), lambda i: (i, 0)),
            pl.BlockSpec((rb, 1), lambda i: (i, 0)),
            pl.BlockSpec((1, D), lambda i: (0, 0)),
            pl.BlockSpec((D, Dn), lambda i: (0, 0)),
        ],
        out_specs=pl.BlockSpec((rb, Dn), lambda i: (i, 0)),
        out_shape=jax.ShapeDtypeStruct((NP, Dn), jnp.float32),
    )(agg, scaled, dinv_col, b_row, Wn)


def _final_sbs(agg, scaled, dinv_col, b_row, rb=2048):
    NP, D = scaled.shape
    npb = NP // rb

    def body(a_ref, s_ref, dv_ref, b_ref, o_ref):
        a = a_ref[...]
        agg_ = (a[:, :D] + a[:, D:] + s_ref[...]) * dv_ref[...]
        o_ref[...] = jnp.tanh(agg_ + b_ref[...])

    return pl.pallas_call(
        body,
        grid=(npb,),
        in_specs=[
            pl.BlockSpec((rb, 2 * D), lambda i: (i, 0)),
            pl.BlockSpec((rb, D), lambda i: (i, 0)),
            pl.BlockSpec((rb, 1), lambda i: (i, 0)),
            pl.BlockSpec((1, D), lambda i: (0, 0)),
        ],
        out_specs=pl.BlockSpec((rb, D), lambda i: (i, 0)),
        out_shape=jax.ShapeDtypeStruct((NP, D), jnp.float32),
    )(agg, scaled, dinv_col, b_row)


def _final(agg, scaled, dinv_col, b_row, rb=2048):
    NP, D = scaled.shape
    npb = NP // rb

    def body(p0_ref, p1_ref, s_ref, dv_ref, b_ref, o_ref):
        agg_ = (p0_ref[...] + p1_ref[...] + s_ref[...]) * dv_ref[...]
        o_ref[...] = jnp.tanh(agg_ + b_ref[...])

    return pl.pallas_call(
        body,
        grid=(npb,),
        in_specs=[
            pl.BlockSpec((rb, D), lambda i: (i, 0)),
            pl.BlockSpec((rb, D), lambda i: (npb + i, 0)),
            pl.BlockSpec((rb, D), lambda i: (i, 0)),
            pl.BlockSpec((rb, 1), lambda i: (i, 0)),
            pl.BlockSpec((1, D), lambda i: (0, 0)),
        ],
        out_specs=pl.BlockSpec((rb, D), lambda i: (i, 0)),
        out_shape=jax.ShapeDtypeStruct((NP, D), jnp.float32),
    )(agg, agg, scaled, dinv_col, b_row)


def kernel(x, edge_index, W1, b1, W2, b2, W3, b3):
    N, _ = x.shape
    E = edge_index.shape[1]

    ei_flat = edge_index.reshape(2 * E)
    hist = _deg_fn(E, N)(ei_flat)
    NP = _pad16x8(N)
    dinv_col = _dinv(hist).reshape(NP, 1)

    K1, ZR1 = 40, 16
    K2, ZR2 = 80, 128
    scaled1 = _matmul_prescale(x, W1, dinv_col)
    agg1 = _agg_fn(N, W1.shape[1], E, K1, ZR1)(scaled1, ei_flat)
    scaled2 = _combine_next(agg1, scaled1, dinv_col, b1.reshape(1, -1), W2)
    agg2 = _agg_fn(N, W2.shape[1], E, K2, 32, side=True, spt=True)(
        scaled2, ei_flat
    )
    scaled3 = _combine_sbs(agg2, scaled2, dinv_col, b2.reshape(1, -1), W3)
    agg3 = _agg_fn(N, W3.shape[1], E, K2, 32, side=True, spt=True)(
        scaled3, ei_flat
    )
    return _final_sbs(agg3, scaled3, dinv_col, b3.reshape(1, -1))[:N]


# speedup vs baseline: 1.0111x; 1.0111x over previous
import functools

import jax
import jax.numpy as jnp
from jax import lax
from jax.experimental import pallas as pl
from jax.experimental.pallas import tpu as pltpu
from jax.experimental.pallas import tpu_sc as plsc

NC = 2
NS = 16
L = 16
NW = NC * NS


def _sc_mesh():
    return plsc.VectorSubcoreMesh(
        core_axis_name="c", subcore_axis_name="s", num_cores=NC, num_subcores=NS
    )


@functools.lru_cache(maxsize=None)
def _deg_fn(E: int, N: int):
    EPW = E // NW
    NP = _pad16x8(N)
    assert EPW * NW == E and EPW % L == 0 and N % L == 0

    @functools.partial(
        pl.kernel,
        out_type=jax.ShapeDtypeStruct((NW, NP), jnp.float32),
        mesh=_sc_mesh(),
        compiler_params=pltpu.CompilerParams(needs_layout_passes=False),
        scratch_types=[
            pltpu.VMEM((EPW,), jnp.int32),
            pltpu.VMEM((NP,), jnp.float32),
        ],
    )
    def k(ei_hbm, out_hbm, idx_v, hist_v):
        cid = lax.axis_index("c")
        sid = lax.axis_index("s")
        wid = cid * NS + sid
        pltpu.sync_copy(ei_hbm.at[pl.ds(E + wid * EPW, EPW)], idx_v)
        zeros = jnp.zeros((L,), jnp.float32)

        def zbody(i, carry):
            hist_v[pl.ds(i * L, L)] = zeros
            return carry

        lax.fori_loop(0, NP // L, zbody, 0)
        ones = jnp.full((L,), 1.0, jnp.float32)

        def body(i, carry):
            ids = idx_v[pl.ds(i * L, L)]
            plsc.addupdate_scatter(hist_v, [ids], ones)
            return carry

        lax.fori_loop(0, EPW // L, body, 0)
        pltpu.sync_copy(hist_v, out_hbm.at[wid])

    return k


def _pad16x8(n):
    q = NS * 40
    return ((n + q - 1) // q) * q


@functools.lru_cache(maxsize=None)
def _agg_fn(N: int, D: int, E: int, K: int, ZR: int, NB: int = 5, LA: int = 3,
            side: bool = False, spt: bool = False):
    EPW = E // NW
    NCH = EPW // K
    NP = _pad16x8(N)
    RPT = NP // NS
    assert NCH * K == EPW and RPT % ZR == 0 and ZR % 8 == 0 and D % L == 0
    assert NCH % NB == 0 and LA < NB and K % 8 == 0

    out_shape = (NP, NC * D) if side else (NC * NP, D)

    @functools.partial(
        pl.kernel,
        out_type=jax.ShapeDtypeStruct(out_shape, jnp.float32),
        mesh=_sc_mesh(),
        compiler_params=pltpu.CompilerParams(
            needs_layout_passes=False, use_tc_tiling_on_sc=False
        ),
        scratch_types=[
            pltpu.VMEM((EPW,), jnp.int32),
            pltpu.VMEM((NCH, K), jnp.int32),
            pltpu.VMEM((NB, K, D), jnp.float32),
            pltpu.VMEM((ZR, D), jnp.float32),
            pltpu.VMEM_SHARED((NP, D), jnp.float32),
            pltpu.VMEM_SHARED((NP, D) if spt else (8, 8), jnp.float32),
            pltpu.SemaphoreType.DMA,
            pltpu.SemaphoreType.DMA,
            [pltpu.SemaphoreType.DMA] * NB,
            [pltpu.SemaphoreType.DMA] * NB,
        ],
    )
    def k(table, ei_hbm, out_hbm, sidx_v, didx, rows, zbuf, acc, table_sh,
          isem, dsem, gsems, ssems):
        cid = lax.axis_index("c")
        sid = lax.axis_index("s")
        wid = cid * NS + sid
        ebase = wid * EPW

        pltpu.async_copy(ei_hbm.at[pl.ds(ebase, EPW)], sidx_v, isem)

        def dfetch(j, carry):
            pltpu.async_copy(
                ei_hbm.at[pl.ds(E + ebase + j * K, K)], didx.at[j], dsem
            )
            return carry

        lax.fori_loop(0, NCH, dfetch, 0)

        zeros = jnp.zeros((L,), jnp.float32)
        vpr = D // L

        def zb(t, carry):
            zbuf[t // vpr, pl.ds((t % vpr) * L, L)] = zeros
            return carry

        lax.fori_loop(0, ZR * vpr, zb, 0)
        for j in range(RPT // ZR):
            pltpu.sync_copy(zbuf, acc.at[pl.ds(sid * RPT + j * ZR, ZR)])
        if spt:
            pltpu.sync_copy(
                table.at[pl.ds(sid * RPT, RPT)],
                table_sh.at[pl.ds(sid * RPT, RPT)],
            )
        pltpu.make_async_copy(ei_hbm.at[pl.ds(ebase, EPW)], sidx_v, isem).wait()

        def dwait(j, carry):
            pltpu.make_async_copy(
                ei_hbm.at[pl.ds(E + ebase, K)], didx.at[0], dsem
            ).wait()
            return carry

        lax.fori_loop(0, NCH, dwait, 0)
        plsc.subcore_barrier()

        gsrc = table_sh if spt else table

        def start_gather(j, b):
            pltpu.async_copy(
                gsrc.at[sidx_v.at[pl.ds(j * K, K)]], rows.at[b], gsems[b]
            )

        def wait_gather(b):
            pltpu.make_async_copy(
                table.at[pl.ds(0, K)], rows.at[b], gsems[b]
            ).wait()

        def start_scatter(j, b):
            pltpu.async_copy(rows.at[b], acc.at[didx.at[j]], ssems[b], add=True)

        def wait_scatter(b):
            pltpu.make_async_copy(rows.at[b], acc.at[pl.ds(0, K)], ssems[b]).wait()

        for b in range(LA):
            start_gather(b, b)

        def body(i, carry):
            j0 = i * NB
            for b in range(NB):
                j = j0 + b
                wait_gather(b)
                start_scatter(j, b)
                jn = j + LA
                bn = (b + LA) % NB

                @pl.when(jn < NCH)
                def _():
                    @pl.when(jn >= NB)
                    def _():
                        wait_scatter(bn)

                    start_gather(jn, bn)

            return carry

        lax.fori_loop(0, NCH // NB, body, 0)
        for b in range(NB):
            wait_scatter(b)
        plsc.subcore_barrier()
        if side:
            pltpu.sync_copy(
                acc.at[pl.ds(sid * RPT, RPT)],
                out_hbm.at[pl.ds(sid * RPT, RPT), pl.ds(cid * D, D)],
            )
        else:
            pltpu.sync_copy(
                acc.at[pl.ds(sid * RPT, RPT)],
                out_hbm.at[pl.ds(cid * NP + sid * RPT, RPT)],
            )

    return k


def _dinv(hist):
    NWh, N = hist.shape

    def body(h_ref, o_ref):
        s = jnp.sum(h_ref[...], axis=0, keepdims=True) + 1.0
        o_ref[...] = lax.rsqrt(s)

    return pl.pallas_call(
        body, out_shape=jax.ShapeDtypeStruct((1, N), jnp.float32)
    )(hist)


def _matmul_prescale(x, W, dinv_col, rb=1000):
    N, Din = x.shape
    NP = _pad16x8(N)
    Dout = W.shape[1]

    def body(x_ref, w_ref, dv_ref, o_ref):
        h = jnp.dot(x_ref[...], w_ref[...], preferred_element_type=jnp.float32)
        o_ref[...] = h * dv_ref[...]

    return pl.pallas_call(
        body,
        grid=(N // rb,),
        in_specs=[
            pl.BlockSpec((rb, Din), lambda i: (i, 0)),
            pl.BlockSpec((Din, Dout), lambda i: (0, 0)),
            pl.BlockSpec((rb, 1), lambda i: (i, 0)),
        ],
        out_specs=pl.BlockSpec((rb, Dout), lambda i: (i, 0)),
        out_shape=jax.ShapeDtypeStruct((NP, Dout), jnp.float32),
    )(x, W, dinv_col)


def _combine_next(agg, scaled, dinv_col, b_row, Wn, rb=1024):
    NP, D = scaled.shape
    npb = NP // rb
    Dn = Wn.shape[1]

    def body(p0_ref, p1_ref, s_ref, dv_ref, b_ref, w_ref, o_ref):
        agg_ = (p0_ref[...] + p1_ref[...] + s_ref[...]) * dv_ref[...]
        act = jnp.tanh(agg_ + b_ref[...])
        h = jnp.dot(act, w_ref[...], preferred_element_type=jnp.float32)
        o_ref[...] = h * dv_ref[...]

    return pl.pallas_call(
        body,
        grid=(npb,),
        in_specs=[
            pl.BlockSpec((rb, D), lambda i: (i, 0)),
            pl.BlockSpec((rb, D), lambda i: (npb + i, 0)),
            pl.BlockSpec((rb, D), lambda i: (i, 0)),
            pl.BlockSpec((rb, 1), lambda i: (i, 0)),
            pl.BlockSpec((1, D), lambda i: (0, 0)),
            pl.BlockSpec((D, Dn), lambda i: (0, 0)),
        ],
        out_specs=pl.BlockSpec((rb, Dn), lambda i: (i, 0)),
        out_shape=jax.ShapeDtypeStruct((NP, Dn), jnp.float32),
    )(agg, agg, scaled, dinv_col, b_row, Wn)


def _combine_sbs(agg, scaled, dinv_col, b_row, Wn, rb=1024):
    NP, D = scaled.shape
    npb = NP // rb
    Dn = Wn.shape[1]

    def body(a_ref, s_ref, dv_ref, b_ref, w_ref, o_ref):
        a = a_ref[...]
        agg_ = (a[:, :D] + a[:, D:] + s_ref[...]) * dv_ref[...]
        act = jnp.tanh(agg_ + b_ref[...])
        h = jnp.dot(act, w_ref[...], preferred_element_type=jnp.float32)
        o_ref[...] = h * dv_ref[...]

    return pl.pallas_call(
        body,
        grid=(npb,),
        in_specs=[
            pl.BlockSpec((rb, 2 * D), lambda i: (i, 0)),
            pl.BlockSpec((rb, D), lambda i: (i, 0)),
            pl.BlockSpec((rb, 1), lambda i: (i, 0)),
            pl.BlockSpec((1, D), lambda i: (0, 0)),
            pl.BlockSpec((D, Dn), lambda i: (0, 0)),
        ],
        out_specs=pl.BlockSpec((rb, Dn), lambda i: (i, 0)),
        out_shape=jax.ShapeDtypeStruct((NP, Dn), jnp.float32),
    )(agg, scaled, dinv_col, b_row, Wn)


def _final_sbs(agg, scaled, dinv_col, b_row, rb=1024):
    NP, D = scaled.shape
    npb = NP // rb

    def body(a_ref, s_ref, dv_ref, b_ref, o_ref):
        a = a_ref[...]
        agg_ = (a[:, :D] + a[:, D:] + s_ref[...]) * dv_ref[...]
        o_ref[...] = jnp.tanh(agg_ + b_ref[...])

    return pl.pallas_call(
        body,
        grid=(npb,),
        in_specs=[
            pl.BlockSpec((rb, 2 * D), lambda i: (i, 0)),
            pl.BlockSpec((rb, D), lambda i: (i, 0)),
            pl.BlockSpec((rb, 1), lambda i: (i, 0)),
            pl.BlockSpec((1, D), lambda i: (0, 0)),
        ],
        out_specs=pl.BlockSpec((rb, D), lambda i: (i, 0)),
        out_shape=jax.ShapeDtypeStruct((NP, D), jnp.float32),
    )(agg, scaled, dinv_col, b_row)


def _final(agg, scaled, dinv_col, b_row, rb=2048):
    NP, D = scaled.shape
    npb = NP // rb

    def body(p0_ref, p1_ref, s_ref, dv_ref, b_ref, o_ref):
        agg_ = (p0_ref[...] + p1_ref[...] + s_ref[...]) * dv_ref[...]
        o_ref[...] = jnp.tanh(agg_ + b_ref[...])

    return pl.pallas_call(
        body,
        grid=(npb,),
        in_specs=[
            pl.BlockSpec((rb, D), lambda i: (i, 0)),
            pl.BlockSpec((rb, D), lambda i: (npb + i, 0)),
            pl.BlockSpec((rb, D), lambda i: (i, 0)),
            pl.BlockSpec((rb, 1), lambda i: (i, 0)),
            pl.BlockSpec((1, D), lambda i: (0, 0)),
        ],
        out_specs=pl.BlockSpec((rb, D), lambda i: (i, 0)),
        out_shape=jax.ShapeDtypeStruct((NP, D), jnp.float32),
    )(agg, agg, scaled, dinv_col, b_row)


def kernel(x, edge_index, W1, b1, W2, b2, W3, b3):
    N, _ = x.shape
    E = edge_index.shape[1]

    ei_flat = edge_index.reshape(2 * E)
    hist = _deg_fn(E, N)(ei_flat)
    NP = _pad16x8(N)
    dinv_col = _dinv(hist).reshape(NP, 1)

    K1, ZR1 = 40, 16
    K2, ZR2 = 80, 128
    scaled1 = _matmul_prescale(x, W1, dinv_col)
    agg1 = _agg_fn(N, W1.shape[1], E, K1, ZR1)(scaled1, ei_flat)
    scaled2 = _combine_next(agg1, scaled1, dinv_col, b1.reshape(1, -1), W2)
    agg2 = _agg_fn(N, W2.shape[1], E, K2, ZR2, side=True)(scaled2, ei_flat)
    scaled3 = _combine_sbs(agg2, scaled2, dinv_col, b2.reshape(1, -1), W3)
    agg3 = _agg_fn(N, W3.shape[1], E, K2, ZR2, side=True)(scaled3, ei_flat)
    return _final_sbs(agg3, scaled3, dinv_col, b3.reshape(1, -1))[:N]


# speedup vs baseline: 1.0570x; 1.0454x over previous
import functools

import jax
import jax.numpy as jnp
from jax import lax
from jax.experimental import pallas as pl
from jax.experimental.pallas import tpu as pltpu
from jax.experimental.pallas import tpu_sc as plsc

NC = 2
NS = 16
L = 16
NW = NC * NS


def _sc_mesh():
    return plsc.VectorSubcoreMesh(
        core_axis_name="c", subcore_axis_name="s", num_cores=NC, num_subcores=NS
    )


@functools.lru_cache(maxsize=None)
def _deg_fn(E: int, N: int):
    EPW = E // NW
    NP = _pad16x8(N)
    assert EPW * NW == E and EPW % L == 0 and N % L == 0

    @functools.partial(
        pl.kernel,
        out_type=jax.ShapeDtypeStruct((NW, NP), jnp.float32),
        mesh=_sc_mesh(),
        compiler_params=pltpu.CompilerParams(needs_layout_passes=False),
        scratch_types=[
            pltpu.VMEM((EPW,), jnp.int32),
            pltpu.VMEM((NP,), jnp.float32),
        ],
    )
    def k(ei_hbm, out_hbm, idx_v, hist_v):
        cid = lax.axis_index("c")
        sid = lax.axis_index("s")
        wid = cid * NS + sid
        pltpu.sync_copy(ei_hbm.at[pl.ds(E + wid * EPW, EPW)], idx_v)
        zeros = jnp.zeros((L,), jnp.float32)

        def zbody(i, carry):
            hist_v[pl.ds(i * L, L)] = zeros
            return carry

        lax.fori_loop(0, NP // L, zbody, 0)
        ones = jnp.full((L,), 1.0, jnp.float32)

        def body(i, carry):
            ids = idx_v[pl.ds(i * L, L)]
            plsc.addupdate_scatter(hist_v, [ids], ones)
            return carry

        lax.fori_loop(0, EPW // L, body, 0)
        pltpu.sync_copy(hist_v, out_hbm.at[wid])

    return k


def _pad16x8(n):
    q = NS * 40
    return ((n + q - 1) // q) * q


@functools.lru_cache(maxsize=None)
def _agg_fn(N: int, D: int, E: int, K: int, ZR: int, NB: int = 5, LA: int = 3,
            side: bool = False, spt: bool = False):
    EPW = E // NW
    NCH = EPW // K
    NP = _pad16x8(N)
    RPT = NP // NS
    assert NCH * K == EPW and RPT % ZR == 0 and ZR % 8 == 0 and D % L == 0
    assert NCH % NB == 0 and LA < NB and K % 8 == 0

    out_shape = (NP, NC * D) if side else (NC * NP, D)

    @functools.partial(
        pl.kernel,
        out_type=jax.ShapeDtypeStruct(out_shape, jnp.float32),
        mesh=_sc_mesh(),
        compiler_params=pltpu.CompilerParams(
            needs_layout_passes=False, use_tc_tiling_on_sc=False
        ),
        scratch_types=[
            pltpu.VMEM((NCH, K), jnp.int32),
            pltpu.VMEM((NCH, K), jnp.int32),
            pltpu.VMEM((NB, K, D), jnp.float32),
            pltpu.VMEM((ZR, D), jnp.float32),
            pltpu.VMEM_SHARED((NP, D), jnp.float32),
            pltpu.VMEM_SHARED((NP, D) if spt else (8, 8), jnp.float32),
            pltpu.SemaphoreType.DMA,
            pltpu.SemaphoreType.DMA,
            [pltpu.SemaphoreType.DMA] * NB,
            [pltpu.SemaphoreType.DMA] * NB,
        ],
    )
    def k(table, ei2d_hbm, out_hbm, sidx_v, didx, rows, zbuf, acc, table_sh,
          isem, dsem, gsems, ssems):
        cid = lax.axis_index("c")
        sid = lax.axis_index("s")
        wid = cid * NS + sid
        ebase = wid * EPW

        srow = ebase // K
        drow = (E + ebase) // K
        pltpu.async_copy(ei2d_hbm.at[pl.ds(srow, NCH)], sidx_v, isem)
        pltpu.async_copy(ei2d_hbm.at[pl.ds(drow, NCH)], didx, dsem)

        zeros = jnp.zeros((L,), jnp.float32)
        vpr = D // L

        def zb(t, carry):
            zbuf[t // vpr, pl.ds((t % vpr) * L, L)] = zeros
            return carry

        lax.fori_loop(0, ZR * vpr, zb, 0)
        for j in range(RPT // ZR):
            pltpu.sync_copy(zbuf, acc.at[pl.ds(sid * RPT + j * ZR, ZR)])
        if spt:
            pltpu.sync_copy(
                table.at[pl.ds(sid * RPT, RPT)],
                table_sh.at[pl.ds(sid * RPT, RPT)],
            )
        pltpu.make_async_copy(ei2d_hbm.at[pl.ds(srow, NCH)], sidx_v, isem).wait()
        pltpu.make_async_copy(ei2d_hbm.at[pl.ds(drow, NCH)], didx, dsem).wait()
        plsc.subcore_barrier()

        gsrc = table_sh if spt else table

        def start_gather(j, b):
            pltpu.async_copy(
                gsrc.at[sidx_v.at[j]], rows.at[b], gsems[b]
            )

        def wait_gather(b):
            pltpu.make_async_copy(
                table.at[pl.ds(0, K)], rows.at[b], gsems[b]
            ).wait()

        def start_scatter(j, b):
            pltpu.async_copy(rows.at[b], acc.at[didx.at[j]], ssems[b], add=True)

        def wait_scatter(b):
            pltpu.make_async_copy(rows.at[b], acc.at[pl.ds(0, K)], ssems[b]).wait()

        for b in range(LA):
            start_gather(b, b)

        def body(i, carry):
            j0 = i * NB
            for b in range(NB):
                j = j0 + b
                wait_gather(b)
                start_scatter(j, b)
                jn = j + LA
                bn = (b + LA) % NB

                @pl.when(jn < NCH)
                def _():
                    @pl.when(jn >= NB)
                    def _():
                        wait_scatter(bn)

                    start_gather(jn, bn)

            return carry

        lax.fori_loop(0, NCH // NB, body, 0)
        for b in range(NB):
            wait_scatter(b)
        plsc.subcore_barrier()
        if side:
            pltpu.sync_copy(
                acc.at[pl.ds(sid * RPT, RPT)],
                out_hbm.at[pl.ds(sid * RPT, RPT), pl.ds(cid * D, D)],
            )
        else:
            pltpu.sync_copy(
                acc.at[pl.ds(sid * RPT, RPT)],
                out_hbm.at[pl.ds(cid * NP + sid * RPT, RPT)],
            )

    return k


def _dinv(hist):
    NWh, N = hist.shape

    def body(h_ref, o_ref):
        s = jnp.sum(h_ref[...], axis=0, keepdims=True) + 1.0
        o_ref[...] = lax.rsqrt(s)

    return pl.pallas_call(
        body, out_shape=jax.ShapeDtypeStruct((1, N), jnp.float32)
    )(hist)


def _matmul_prescale(x, W, dinv_col, rb=2000):
    N, Din = x.shape
    NP = _pad16x8(N)
    Dout = W.shape[1]

    def body(x_ref, w_ref, dv_ref, o_ref):
        h = jnp.dot(x_ref[...], w_ref[...], preferred_element_type=jnp.float32)
        o_ref[...] = h * dv_ref[...]

    return pl.pallas_call(
        body,
        grid=(N // rb,),
        in_specs=[
            pl.BlockSpec((rb, Din), lambda i: (i, 0)),
            pl.BlockSpec((Din, Dout), lambda i: (0, 0)),
            pl.BlockSpec((rb, 1), lambda i: (i, 0)),
        ],
        out_specs=pl.BlockSpec((rb, Dout), lambda i: (i, 0)),
        out_shape=jax.ShapeDtypeStruct((NP, Dout), jnp.float32),
    )(x, W, dinv_col)


def _combine_next(agg, scaled, dinv_col, b_row, Wn, rb=2048):
    NP, D = scaled.shape
    npb = NP // rb
    Dn = Wn.shape[1]

    def body(p0_ref, p1_ref, s_ref, dv_ref, b_ref, w_ref, o_ref):
        agg_ = (p0_ref[...] + p1_ref[...] + s_ref[...]) * dv_ref[...]
        act = jnp.tanh(agg_ + b_ref[...])
        h = jnp.dot(act, w_ref[...], preferred_element_type=jnp.float32)
        o_ref[...] = h * dv_ref[...]

    return pl.pallas_call(
        body,
        grid=(npb,),
        in_specs=[
            pl.BlockSpec((rb, D), lambda i: (i, 0)),
            pl.BlockSpec((rb, D), lambda i: (npb + i, 0)),
            pl.BlockSpec((rb, D), lambda i: (i, 0)),
            pl.BlockSpec((rb, 1), lambda i: (i, 0)),
            pl.BlockSpec((1, D), lambda i: (0, 0)),
            pl.BlockSpec((D, Dn), lambda i: (0, 0)),
        ],
        out_specs=pl.BlockSpec((rb, Dn), lambda i: (i, 0)),
        out_shape=jax.ShapeDtypeStruct((NP, Dn), jnp.float32),
    )(agg, agg, scaled, dinv_col, b_row, Wn)


def _combine_sbs(agg, scaled, dinv_col, b_row, Wn, rb=2048):
    NP, D = scaled.shape
    npb = NP // rb
    Dn = Wn.shape[1]

    def body(a_ref, s_ref, dv_ref, b_ref, w_ref, o_ref):
        a = a_ref[...]
        agg_ = (a[:, :D] + a[:, D:] + s_ref[...]) * dv_ref[...]
        act = jnp.tanh(agg_ + b_ref[...])
        h = jnp.dot(act, w_ref[...], preferred_element_type=jnp.float32)
        o_ref[...] = h * dv_ref[...]

    return pl.pallas_call(
        body,
        grid=(npb,),
        in_specs=[
            pl.BlockSpec((rb, 2 * D), lambda i: (i, 0)),
            pl.BlockSpec((rb, D), lambda i: (i, 0)),
            pl.BlockSpec((rb, 1), lambda i: (i, 0)),
            pl.BlockSpec((1, D), lambda i: (0, 0)),
            pl.BlockSpec((D, Dn), lambda i: (0, 0)),
        ],
        out_specs=pl.BlockSpec((rb, Dn), lambda i: (i, 0)),
        out_shape=jax.ShapeDtypeStruct((NP, Dn), jnp.float32),
    )(agg, scaled, dinv_col, b_row, Wn)


def _final_sbs(agg, scaled, dinv_col, b_row, rb=2048):
    NP, D = scaled.shape
    npb = NP // rb

    def body(a_ref, s_ref, dv_ref, b_ref, o_ref):
        a = a_ref[...]
        agg_ = (a[:, :D] + a[:, D:] + s_ref[...]) * dv_ref[...]
        o_ref[...] = jnp.tanh(agg_ + b_ref[...])

    return pl.pallas_call(
        body,
        grid=(npb,),
        in_specs=[
            pl.BlockSpec((rb, 2 * D), lambda i: (i, 0)),
            pl.BlockSpec((rb, D), lambda i: (i, 0)),
            pl.BlockSpec((rb, 1), lambda i: (i, 0)),
            pl.BlockSpec((1, D), lambda i: (0, 0)),
        ],
        out_specs=pl.BlockSpec((rb, D), lambda i: (i, 0)),
        out_shape=jax.ShapeDtypeStruct((NP, D), jnp.float32),
    )(agg, scaled, dinv_col, b_row)


def _final(agg, scaled, dinv_col, b_row, rb=2048):
    NP, D = scaled.shape
    npb = NP // rb

    def body(p0_ref, p1_ref, s_ref, dv_ref, b_ref, o_ref):
        agg_ = (p0_ref[...] + p1_ref[...] + s_ref[...]) * dv_ref[...]
        o_ref[...] = jnp.tanh(agg_ + b_ref[...])

    return pl.pallas_call(
        body,
        grid=(npb,),
        in_specs=[
            pl.BlockSpec((rb, D), lambda i: (i, 0)),
            pl.BlockSpec((rb, D), lambda i: (npb + i, 0)),
            pl.BlockSpec((rb, D), lambda i: (i, 0)),
            pl.BlockSpec((rb, 1), lambda i: (i, 0)),
            pl.BlockSpec((1, D), lambda i: (0, 0)),
        ],
        out_specs=pl.BlockSpec((rb, D), lambda i: (i, 0)),
        out_shape=jax.ShapeDtypeStruct((NP, D), jnp.float32),
    )(agg, agg, scaled, dinv_col, b_row)


def kernel(x, edge_index, W1, b1, W2, b2, W3, b3):
    N, _ = x.shape
    E = edge_index.shape[1]

    ei_flat = edge_index.reshape(2 * E)
    hist = _deg_fn(E, N)(ei_flat)
    NP = _pad16x8(N)
    dinv_col = _dinv(hist).reshape(NP, 1)

    K1, ZR1 = 40, 16
    K2, ZR2 = 80, 128
    ei2d_a = ei_flat.reshape(2 * E // K1, K1)
    ei2d_b = ei_flat.reshape(2 * E // K2, K2)
    scaled1 = _matmul_prescale(x, W1, dinv_col)
    agg1 = _agg_fn(N, W1.shape[1], E, K1, ZR1)(scaled1, ei2d_a)
    scaled2 = _combine_next(agg1, scaled1, dinv_col, b1.reshape(1, -1), W2)
    agg2 = _agg_fn(N, W2.shape[1], E, K2, ZR2, side=True)(scaled2, ei2d_b)
    scaled3 = _combine_sbs(agg2, scaled2, dinv_col, b2.reshape(1, -1), W3)
    agg3 = _agg_fn(N, W3.shape[1], E, K2, ZR2, side=True)(scaled3, ei2d_b)
    return _final_sbs(agg3, scaled3, dinv_col, b3.reshape(1, -1))[:N]


# speedup vs baseline: 1.0916x; 1.0328x over previous
import functools

import jax
import jax.numpy as jnp
from jax import lax
from jax.experimental import pallas as pl
from jax.experimental.pallas import tpu as pltpu
from jax.experimental.pallas import tpu_sc as plsc

NC = 2
NS = 16
L = 16
NW = NC * NS


def _sc_mesh():
    return plsc.VectorSubcoreMesh(
        core_axis_name="c", subcore_axis_name="s", num_cores=NC, num_subcores=NS
    )


@functools.lru_cache(maxsize=None)
def _deg_fn(E: int, N: int):
    EPW = E // NW
    NP = _pad16x8(N)
    assert EPW * NW == E and EPW % L == 0 and N % L == 0

    @functools.partial(
        pl.kernel,
        out_type=jax.ShapeDtypeStruct((NW, NP), jnp.float32),
        mesh=_sc_mesh(),
        compiler_params=pltpu.CompilerParams(needs_layout_passes=False),
        scratch_types=[
            pltpu.VMEM((EPW,), jnp.int32),
            pltpu.VMEM((NP,), jnp.float32),
        ],
    )
    def k(ei_hbm, out_hbm, idx_v, hist_v):
        cid = lax.axis_index("c")
        sid = lax.axis_index("s")
        wid = cid * NS + sid
        pltpu.sync_copy(ei_hbm.at[pl.ds(E + wid * EPW, EPW)], idx_v)
        zeros = jnp.zeros((L,), jnp.float32)

        def zbody(i, carry):
            hist_v[pl.ds(i * L, L)] = zeros
            return carry

        lax.fori_loop(0, NP // L, zbody, 0)
        ones = jnp.full((L,), 1.0, jnp.float32)

        def body(i, carry):
            ids = idx_v[pl.ds(i * L, L)]
            plsc.addupdate_scatter(hist_v, [ids], ones)
            return carry

        lax.fori_loop(0, EPW // L, body, 0)
        pltpu.sync_copy(hist_v, out_hbm.at[wid])

    return k


def _pad16x8(n):
    q = NS * 40
    return ((n + q - 1) // q) * q


@functools.lru_cache(maxsize=None)
def _agg_fn(N: int, D: int, E: int, K: int, ZR: int, NB: int = 5, LA: int = 3,
            side: bool = False, spt: bool = False):
    EPW = E // NW
    NCH = EPW // K
    NP = _pad16x8(N)
    RPT = NP // NS
    assert NCH * K == EPW and RPT % ZR == 0 and ZR % 8 == 0 and D % L == 0
    assert NCH % NB == 0 and LA < NB and K % 8 == 0

    out_shape = (NP, NC * D) if side else (NC * NP, D)

    @functools.partial(
        pl.kernel,
        out_type=jax.ShapeDtypeStruct(out_shape, jnp.float32),
        mesh=_sc_mesh(),
        compiler_params=pltpu.CompilerParams(
            needs_layout_passes=False, use_tc_tiling_on_sc=False
        ),
        scratch_types=[
            pltpu.VMEM((NCH, K), jnp.int32),
            pltpu.VMEM((NCH, K), jnp.int32),
            pltpu.VMEM((NB, K, D), jnp.float32),
            pltpu.VMEM((ZR, D), jnp.float32),
            pltpu.VMEM_SHARED((NP, D), jnp.float32),
            pltpu.VMEM_SHARED((NP, D) if spt else (8, 8), jnp.float32),
            pltpu.SemaphoreType.DMA,
            pltpu.SemaphoreType.DMA,
            [pltpu.SemaphoreType.DMA] * NB,
            [pltpu.SemaphoreType.DMA] * NB,
        ],
    )
    def k(table, ei2d_hbm, out_hbm, sidx_v, didx, rows, zbuf, acc, table_sh,
          isem, dsem, gsems, ssems):
        cid = lax.axis_index("c")
        sid = lax.axis_index("s")
        wid = cid * NS + sid
        ebase = wid * EPW

        srow = ebase // K
        drow = (E + ebase) // K
        pltpu.async_copy(ei2d_hbm.at[pl.ds(srow, NCH)], sidx_v, isem)
        pltpu.async_copy(ei2d_hbm.at[pl.ds(drow, NCH)], didx, dsem)

        zeros = jnp.zeros((L,), jnp.float32)
        vpr = D // L

        def zb(t, carry):
            zbuf[t // vpr, pl.ds((t % vpr) * L, L)] = zeros
            return carry

        lax.fori_loop(0, ZR * vpr, zb, 0)
        for j in range(RPT // ZR):
            pltpu.sync_copy(zbuf, acc.at[pl.ds(sid * RPT + j * ZR, ZR)])
        if spt:
            pltpu.sync_copy(
                table.at[pl.ds(sid * RPT, RPT)],
                table_sh.at[pl.ds(sid * RPT, RPT)],
            )
        pltpu.make_async_copy(ei2d_hbm.at[pl.ds(srow, NCH)], sidx_v, isem).wait()
        pltpu.make_async_copy(ei2d_hbm.at[pl.ds(drow, NCH)], didx, dsem).wait()
        plsc.subcore_barrier()

        gsrc = table_sh if spt else table

        def start_gather(j, b):
            pltpu.async_copy(
                gsrc.at[sidx_v.at[j]], rows.at[b], gsems[b]
            )

        def wait_gather(b):
            pltpu.make_async_copy(
                table.at[pl.ds(0, K)], rows.at[b], gsems[b]
            ).wait()

        def start_scatter(j, b):
            pltpu.async_copy(rows.at[b], acc.at[didx.at[j]], ssems[b], add=True)

        def wait_scatter(b):
            pltpu.make_async_copy(rows.at[b], acc.at[pl.ds(0, K)], ssems[b]).wait()

        for b in range(LA):
            start_gather(b, b)

        def body(i, carry):
            j0 = i * NB
            for b in range(NB):
                j = j0 + b
                wait_gather(b)
                start_scatter(j, b)
                jn = j + LA
                bn = (b + LA) % NB

                @pl.when(jn < NCH)
                def _():
                    @pl.when(jn >= NB)
                    def _():
                        wait_scatter(bn)

                    start_gather(jn, bn)

            return carry

        lax.fori_loop(0, NCH // NB, body, 0)
        for b in range(NB):
            wait_scatter(b)
        plsc.subcore_barrier()
        if side:
            pltpu.sync_copy(
                acc.at[pl.ds(sid * RPT, RPT)],
                out_hbm.at[pl.ds(sid * RPT, RPT), pl.ds(cid * D, D)],
            )
        else:
            pltpu.sync_copy(
                acc.at[pl.ds(sid * RPT, RPT)],
                out_hbm.at[pl.ds(cid * NP + sid * RPT, RPT)],
            )

    return k


def _dinv(hist):
    NWh, N = hist.shape

    def body(h_ref, o_ref):
        s = jnp.sum(h_ref[...], axis=0, keepdims=True) + 1.0
        o_ref[...] = lax.rsqrt(s)

    return pl.pallas_call(
        body, out_shape=jax.ShapeDtypeStruct((1, N), jnp.float32)
    )(hist)


def _matmul_prescale(x, W, dinv_col, rb=2000):
    N, Din = x.shape
    NP = _pad16x8(N)
    Dout = W.shape[1]

    def body(x_ref, w_ref, dv_ref, o_ref):
        h = jnp.dot(x_ref[...], w_ref[...], preferred_element_type=jnp.float32)
        o_ref[...] = h * dv_ref[...]

    return pl.pallas_call(
        body,
        grid=(N // rb,),
        in_specs=[
            pl.BlockSpec((rb, Din), lambda i: (i, 0)),
            pl.BlockSpec((Din, Dout), lambda i: (0, 0)),
            pl.BlockSpec((rb, 1), lambda i: (i, 0)),
        ],
        out_specs=pl.BlockSpec((rb, Dout), lambda i: (i, 0)),
        out_shape=jax.ShapeDtypeStruct((NP, Dout), jnp.float32),
    )(x, W, dinv_col)


def _combine_next(agg, scaled, dinv_col, b_row, Wn, rb=2048):
    NP, D = scaled.shape
    npb = NP // rb
    Dn = Wn.shape[1]

    def body(p0_ref, p1_ref, s_ref, dv_ref, b_ref, w_ref, o_ref):
        agg_ = (p0_ref[...] + p1_ref[...] + s_ref[...]) * dv_ref[...]
        act = jnp.tanh(agg_ + b_ref[...])
        h = jnp.dot(act, w_ref[...], preferred_element_type=jnp.float32)
        o_ref[...] = h * dv_ref[...]

    return pl.pallas_call(
        body,
        grid=(npb,),
        in_specs=[
            pl.BlockSpec((rb, D), lambda i: (i, 0)),
            pl.BlockSpec((rb, D), lambda i: (npb + i, 0)),
            pl.BlockSpec((rb, D), lambda i: (i, 0)),
            pl.BlockSpec((rb, 1), lambda i: (i, 0)),
            pl.BlockSpec((1, D), lambda i: (0, 0)),
            pl.BlockSpec((D, Dn), lambda i: (0, 0)),
        ],
        out_specs=pl.BlockSpec((rb, Dn), lambda i: (i, 0)),
        out_shape=jax.ShapeDtypeStruct((NP, Dn), jnp.float32),
    )(agg, agg, scaled, dinv_col, b_row, Wn)


def _combine_sbs(agg, scaled, dinv_col, b_row, Wn, rb=2048):
    NP, D = scaled.shape
    npb = NP // rb
    Dn = Wn.shape[1]

    def body(a_ref, s_ref, dv_ref, b_ref, w_ref, o_ref):
        a = a_ref[...]
        agg_ = (a[:, :D] + a[:, D:] + s_ref[...]) * dv_ref[...]
        act = jnp.tanh(agg_ + b_ref[...])
        h = jnp.dot(act, w_ref[...], preferred_element_type=jnp.float32)
        o_ref[...] = h * dv_ref[...]

    return pl.pallas_call(
        body,
        grid=(npb,),
        in_specs=[
            pl.BlockSpec((rb, 2 * D), lambda i: (i, 0)),
            pl.BlockSpec((rb, D), lambda i: (i, 0)),
            pl.BlockSpec((rb, 1), lambda i: (i, 0)),
            pl.BlockSpec((1, D), lambda i: (0, 0)),
            pl.BlockSpec((D, Dn), lambda i: (0, 0)),
        ],
        out_specs=pl.BlockSpec((rb, Dn), lambda i: (i, 0)),
        out_shape=jax.ShapeDtypeStruct((NP, Dn), jnp.float32),
    )(agg, scaled, dinv_col, b_row, Wn)


def _final_sbs(agg, scaled, dinv_col, b_row, rb=2048):
    NP, D = scaled.shape
    npb = NP // rb

    def body(a_ref, s_ref, dv_ref, b_ref, o_ref):
        a = a_ref[...]
        agg_ = (a[:, :D] + a[:, D:] + s_ref[...]) * dv_ref[...]
        o_ref[...] = jnp.tanh(agg_ + b_ref[...])

    return pl.pallas_call(
        body,
        grid=(npb,),
        in_specs=[
            pl.BlockSpec((rb, 2 * D), lambda i: (i, 0)),
            pl.BlockSpec((rb, D), lambda i: (i, 0)),
            pl.BlockSpec((rb, 1), lambda i: (i, 0)),
            pl.BlockSpec((1, D), lambda i: (0, 0)),
        ],
        out_specs=pl.BlockSpec((rb, D), lambda i: (i, 0)),
        out_shape=jax.ShapeDtypeStruct((NP, D), jnp.float32),
    )(agg, scaled, dinv_col, b_row)


def _final(agg, scaled, dinv_col, b_row, rb=2048):
    NP, D = scaled.shape
    npb = NP // rb

    def body(p0_ref, p1_ref, s_ref, dv_ref, b_ref, o_ref):
        agg_ = (p0_ref[...] + p1_ref[...] + s_ref[...]) * dv_ref[...]
        o_ref[...] = jnp.tanh(agg_ + b_ref[...])

    return pl.pallas_call(
        body,
        grid=(npb,),
        in_specs=[
            pl.BlockSpec((rb, D), lambda i: (i, 0)),
            pl.BlockSpec((rb, D), lambda i: (npb + i, 0)),
            pl.BlockSpec((rb, D), lambda i: (i, 0)),
            pl.BlockSpec((rb, 1), lambda i: (i, 0)),
            pl.BlockSpec((1, D), lambda i: (0, 0)),
        ],
        out_specs=pl.BlockSpec((rb, D), lambda i: (i, 0)),
        out_shape=jax.ShapeDtypeStruct((NP, D), jnp.float32),
    )(agg, agg, scaled, dinv_col, b_row)


def kernel(x, edge_index, W1, b1, W2, b2, W3, b3):
    N, _ = x.shape
    E = edge_index.shape[1]

    ei_flat = edge_index.reshape(2 * E)
    hist = _deg_fn(E, N)(ei_flat)
    NP = _pad16x8(N)
    dinv_col = _dinv(hist).reshape(NP, 1)

    K1, ZR1 = 40, 16
    K2, ZR2 = 40, 128
    ei2d_a = ei_flat.reshape(2 * E // K1, K1)
    ei2d_b = ei_flat.reshape(2 * E // K2, K2)
    scaled1 = _matmul_prescale(x, W1, dinv_col)
    agg1 = _agg_fn(N, W1.shape[1], E, K1, ZR1, NB=5, LA=4)(scaled1, ei2d_a)
    scaled2 = _combine_next(agg1, scaled1, dinv_col, b1.reshape(1, -1), W2)
    agg2 = _agg_fn(N, W2.shape[1], E, K2, ZR2, NB=10, LA=6, side=True)(scaled2, ei2d_b)
    scaled3 = _combine_sbs(agg2, scaled2, dinv_col, b2.reshape(1, -1), W3)
    agg3 = _agg_fn(N, W3.shape[1], E, K2, ZR2, NB=10, LA=6, side=True)(scaled3, ei2d_b)
    return _final_sbs(agg3, scaled3, dinv_col, b3.reshape(1, -1))[:N]
